# Initial kernel scaffold; baseline (speedup 1.0000x reference)
import functools

import jax
import jax.numpy as jnp
from jax import lax
from jax.experimental import pallas as pl
from jax.experimental.pallas import tpu as pltpu
from jax.experimental.pallas import tpu_sc as plsc

N = 10000
E = 160000
D_IN = 1433
H1 = 32
H2 = 16

NC = 2
NS = 16
NW = NC * NS
LANES = 16
CHUNK = 128
NCHUNKS = E // CHUNK
CHUNKS_PER_TILE = (NCHUNKS + NW - 1) // NW
ROWS_PER_TILE = N // NS


def _spmm_sc(y, src, dst, w):
    f = y.shape[1]
    mesh = plsc.VectorSubcoreMesh(
        core_axis_name="c", subcore_axis_name="s", num_cores=NC, num_subcores=NS
    )
    zeros = jnp.zeros((N, f), jnp.float32)

    @functools.partial(
        pl.kernel,
        out_type=jax.ShapeDtypeStruct((NC, N, f), jnp.float32),
        mesh=mesh,
        scratch_types=[
            pltpu.VMEM((CHUNK,), jnp.int32),
            pltpu.VMEM((CHUNK,), jnp.int32),
            pltpu.VMEM((CHUNK,), jnp.float32),
            pltpu.VMEM((CHUNK, f), jnp.float32),
            pltpu.VMEM_SHARED((N, f), jnp.float32),
            pltpu.SemaphoreType.DMA,
        ],
    )
    def k(y_hbm, src_hbm, dst_hbm, w_hbm, z_hbm, out_hbm,
          idx_s, idx_d, wbuf, rows, acc, sem):
        cid = lax.axis_index("c")
        sid = lax.axis_index("s")
        wid = sid * NC + cid

        pltpu.sync_copy(
            z_hbm.at[pl.ds(sid * ROWS_PER_TILE, ROWS_PER_TILE)],
            acc.at[pl.ds(sid * ROWS_PER_TILE, ROWS_PER_TILE)],
        )
        plsc.subcore_barrier()

        def chunk_body(i, carry):
            kk = wid + NW * i

            @pl.when(kk < NCHUNKS)
            def _():
                base = kk * CHUNK
                pltpu.sync_copy(src_hbm.at[pl.ds(base, CHUNK)], idx_s)
                pltpu.sync_copy(dst_hbm.at[pl.ds(base, CHUNK)], idx_d)
                pltpu.sync_copy(w_hbm.at[pl.ds(base, CHUNK)], wbuf)
                pltpu.async_copy(y_hbm.at[idx_s], rows, sem).wait()

                def edge_body(j, c2):
                    wsplat = plsc.load_gather(
                        wbuf, [jnp.zeros((LANES,), jnp.int32) + j]
                    )
                    for fb in range(f // LANES):
                        v = rows[j, pl.ds(fb * LANES, LANES)]
                        rows[j, pl.ds(fb * LANES, LANES)] = v * wsplat
                    return c2

                lax.fori_loop(0, CHUNK, edge_body, 0)
                pltpu.sync_copy(rows, acc.at[idx_d], add=True)

            return carry

        lax.fori_loop(0, CHUNKS_PER_TILE, chunk_body, 0)
        plsc.subcore_barrier()

        pltpu.sync_copy(
            acc.at[pl.ds(sid * ROWS_PER_TILE, ROWS_PER_TILE)],
            out_hbm.at[cid, pl.ds(sid * ROWS_PER_TILE, ROWS_PER_TILE)],
        )

    return k(y, src, dst, w, zeros)


def _mm1(x, w1):
    bm = 1000

    def body(x_ref, w_ref, o_ref):
        o_ref[...] = jnp.dot(x_ref[...], w_ref[...],
                             preferred_element_type=jnp.float32)

    return pl.pallas_call(
        body,
        grid=(N // bm,),
        in_specs=[
            pl.BlockSpec((bm, D_IN), lambda i: (i, 0)),
            pl.BlockSpec((D_IN, H1), lambda i: (0, 0)),
        ],
        out_specs=pl.BlockSpec((bm, H1), lambda i: (i, 0)),
        out_shape=jax.ShapeDtypeStruct((N, H1), jnp.float32),
    )(x, w1)


def _relu_mm(p, w):
    fin = p.shape[2]
    fout = w.shape[1]
    bm = 2000

    def body(p_ref, w_ref, o_ref):
        h = jax.nn.relu(p_ref[0] + p_ref[1])
        o_ref[...] = jnp.dot(h, w_ref[...], preferred_element_type=jnp.float32)

    return pl.pallas_call(
        body,
        grid=(N // bm,),
        in_specs=[
            pl.BlockSpec((2, bm, fin), lambda i: (0, i, 0)),
            pl.BlockSpec((fin, fout), lambda i: (0, 0)),
        ],
        out_specs=pl.BlockSpec((bm, fout), lambda i: (i, 0)),
        out_shape=jax.ShapeDtypeStruct((N, fout), jnp.float32),
    )(p, w)


def _zsum(p):
    f = p.shape[2]
    bm = 2000

    def body(p_ref, o_ref):
        o_ref[...] = p_ref[0] + p_ref[1]

    return pl.pallas_call(
        body,
        grid=(N // bm,),
        in_specs=[pl.BlockSpec((2, bm, f), lambda i: (0, i, 0))],
        out_specs=pl.BlockSpec((bm, f), lambda i: (i, 0)),
        out_shape=jax.ShapeDtypeStruct((N, f), jnp.float32),
    )(p)


def _decode(z, zt):
    bm, bn = 1000, 1000

    def body(zr_ref, zc_ref, o_ref):
        logits = jnp.dot(zr_ref[...], zc_ref[...],
                         preferred_element_type=jnp.float32)
        o_ref[...] = jax.nn.sigmoid(logits)

    return pl.pallas_call(
        body,
        grid=(N // bm, N // bn),
        in_specs=[
            pl.BlockSpec((bm, H2), lambda i, j: (i, 0)),
            pl.BlockSpec((H2, bn), lambda i, j: (0, j)),
        ],
        out_specs=pl.BlockSpec((bm, bn), lambda i, j: (i, j)),
        out_shape=jax.ShapeDtypeStruct((N, N), jnp.float32),
    )(z, zt)


def kernel(X, edge_index, edge_weight, W1, W2, W3):
    src = edge_index[1]
    dst = edge_index[0]
    y1 = _mm1(X, W1)
    p1 = _spmm_sc(y1, src, dst, edge_weight)
    y2 = _relu_mm(p1, W2)
    p2 = _spmm_sc(y2, src, dst, edge_weight)
    y3 = _relu_mm(p2, W3)
    p3 = _spmm_sc(y3, src, dst, edge_weight)
    z = _zsum(p3)
    a = _decode(z, z.T)
    return (a, z)


# speedup vs baseline: 4.5119x; 4.5119x over previous
import functools

import jax
import jax.numpy as jnp
from jax import lax
from jax.experimental import pallas as pl
from jax.experimental.pallas import tpu as pltpu
from jax.experimental.pallas import tpu_sc as plsc

N = 10000
E = 160000
D_IN = 1433
H1 = 32
H2 = 16

NC = 2
NS = 16
NW = NC * NS
LANES = 16
CHUNK = 128
NCHUNKS = E // CHUNK
CHUNKS_PER_TILE = (NCHUNKS + NW - 1) // NW
RPT = 624
TAIL_BASE = NS * RPT
TAIL = N - TAIL_BASE


def _spmm_sc(y, src, dst, w):
    f = y.shape[1]
    mesh = plsc.VectorSubcoreMesh(
        core_axis_name="c", subcore_axis_name="s", num_cores=NC, num_subcores=NS
    )
    zeros = jnp.zeros((N, f), jnp.float32)

    @functools.partial(
        pl.kernel,
        out_type=jax.ShapeDtypeStruct((NC, N, f), jnp.float32),
        mesh=mesh,
        scratch_types=[
            pltpu.VMEM((CHUNK,), jnp.int32),
            pltpu.VMEM((CHUNK,), jnp.int32),
            pltpu.VMEM((CHUNK,), jnp.float32),
            pltpu.VMEM((CHUNK, f), jnp.float32),
            pltpu.VMEM_SHARED((N, f), jnp.float32),
            pltpu.SemaphoreType.DMA,
        ],
        compiler_params=pltpu.CompilerParams(use_tc_tiling_on_sc=False),
    )
    def k(y_hbm, src_hbm, dst_hbm, w_hbm, z_hbm, out_hbm,
          idx_s, idx_d, wbuf, rows, acc, sem):
        cid = lax.axis_index("c")
        sid = lax.axis_index("s")
        wid = sid * NC + cid

        pltpu.sync_copy(
            z_hbm.at[pl.ds(sid * RPT, RPT)],
            acc.at[pl.ds(sid * RPT, RPT)],
        )

        @pl.when(sid == NS - 1)
        def _():
            pltpu.sync_copy(
                z_hbm.at[pl.ds(TAIL_BASE, TAIL)],
                acc.at[pl.ds(TAIL_BASE, TAIL)],
            )

        plsc.subcore_barrier()

        def chunk_body(i, carry):
            kk = wid + NW * i

            @pl.when(kk < NCHUNKS)
            def _():
                base = kk * CHUNK
                pltpu.sync_copy(src_hbm.at[pl.ds(base, CHUNK)], idx_s)
                pltpu.sync_copy(dst_hbm.at[pl.ds(base, CHUNK)], idx_d)
                pltpu.sync_copy(w_hbm.at[pl.ds(base, CHUNK)], wbuf)
                pltpu.async_copy(y_hbm.at[idx_s], rows, sem).wait()

                def edge_body(g, c2):
                    j0 = g * LANES
                    wv = wbuf[pl.ds(j0, LANES)]
                    for l in range(LANES):
                        wl = wv[l]
                        for fb in range(f // LANES):
                            v = rows[j0 + l, pl.ds(fb * LANES, LANES)]
                            rows[j0 + l, pl.ds(fb * LANES, LANES)] = v * wl
                    return c2

                lax.fori_loop(0, CHUNK // LANES, edge_body, 0)
                pltpu.sync_copy(rows, acc.at[idx_d], add=True)

            return carry

        lax.fori_loop(0, CHUNKS_PER_TILE, chunk_body, 0)
        plsc.subcore_barrier()

        pltpu.sync_copy(
            acc.at[pl.ds(sid * RPT, RPT)],
            out_hbm.at[cid, pl.ds(sid * RPT, RPT)],
        )

        @pl.when(sid == NS - 1)
        def _():
            pltpu.sync_copy(
                acc.at[pl.ds(TAIL_BASE, TAIL)],
                out_hbm.at[cid, pl.ds(TAIL_BASE, TAIL)],
            )

    return k(y, src, dst, w, zeros)


def _mm1(x, w1):
    bm = 1000

    def body(x_ref, w_ref, o_ref):
        o_ref[...] = jnp.dot(x_ref[...], w_ref[...],
                             preferred_element_type=jnp.float32)

    return pl.pallas_call(
        body,
        grid=(N // bm,),
        in_specs=[
            pl.BlockSpec((bm, D_IN), lambda i: (i, 0)),
            pl.BlockSpec((D_IN, H1), lambda i: (0, 0)),
        ],
        out_specs=pl.BlockSpec((bm, H1), lambda i: (i, 0)),
        out_shape=jax.ShapeDtypeStruct((N, H1), jnp.float32),
    )(x, w1)


def _relu_mm(p, w):
    fin = p.shape[2]
    fout = w.shape[1]
    bm = 2000

    def body(p_ref, w_ref, o_ref):
        h = jax.nn.relu(p_ref[0] + p_ref[1])
        o_ref[...] = jnp.dot(h, w_ref[...], preferred_element_type=jnp.float32)

    return pl.pallas_call(
        body,
        grid=(N // bm,),
        in_specs=[
            pl.BlockSpec((2, bm, fin), lambda i: (0, i, 0)),
            pl.BlockSpec((fin, fout), lambda i: (0, 0)),
        ],
        out_specs=pl.BlockSpec((bm, fout), lambda i: (i, 0)),
        out_shape=jax.ShapeDtypeStruct((N, fout), jnp.float32),
    )(p, w)


def _zsum(p):
    f = p.shape[2]
    bm = 2000

    def body(p_ref, o_ref):
        o_ref[...] = p_ref[0] + p_ref[1]

    return pl.pallas_call(
        body,
        grid=(N // bm,),
        in_specs=[pl.BlockSpec((2, bm, f), lambda i: (0, i, 0))],
        out_specs=pl.BlockSpec((bm, f), lambda i: (i, 0)),
        out_shape=jax.ShapeDtypeStruct((N, f), jnp.float32),
    )(p)


def _decode(z, zt):
    bm = 400

    def body(zr_ref, zc_ref, o_ref):
        logits = jnp.dot(zr_ref[...], zc_ref[...],
                         preferred_element_type=jnp.float32)
        o_ref[...] = jax.nn.sigmoid(logits)

    return pl.pallas_call(
        body,
        grid=(N // bm,),
        in_specs=[
            pl.BlockSpec((bm, H2), lambda i: (i, 0)),
            pl.BlockSpec((H2, N), lambda i: (0, 0)),
        ],
        out_specs=pl.BlockSpec((bm, N), lambda i: (i, 0)),
        out_shape=jax.ShapeDtypeStruct((N, N), jnp.float32),
    )(z, zt)


def kernel(X, edge_index, edge_weight, W1, W2, W3):
    src = edge_index[1]
    dst = edge_index[0]
    y1 = _mm1(X, W1)
    p1 = _spmm_sc(y1, src, dst, edge_weight)
    y2 = _relu_mm(p1, W2)
    p2 = _spmm_sc(y2, src, dst, edge_weight)
    y3 = _relu_mm(p2, W3)
    p3 = _spmm_sc(y3, src, dst, edge_weight)
    z = _zsum(p3)
    a = _decode(z, z.T)
    return (a, z)


# speedup vs baseline: 7.0608x; 1.5649x over previous
import functools

import jax
import jax.numpy as jnp
from jax import lax
from jax.experimental import pallas as pl
from jax.experimental.pallas import tpu as pltpu
from jax.experimental.pallas import tpu_sc as plsc

N = 10000
E = 160000
D_IN = 1433
H1 = 32
H2 = 16

NC = 2
NS = 16
NW = NC * NS
LANES = 16
CHUNK = 128
NCHUNKS = E // CHUNK
CPT = NCHUNKS // NW
CPT_LAST = NCHUNKS - (NW - 1) * CPT
RPT = 624
TAIL_BASE = NS * RPT
TAIL = N - TAIL_BASE


def _spmm_sc(y, src, dst, w):
    f = y.shape[1]
    mesh = plsc.VectorSubcoreMesh(
        core_axis_name="c", subcore_axis_name="s", num_cores=NC, num_subcores=NS
    )
    zeros = jnp.zeros((N, f), jnp.float32)
    src2 = src.reshape(NCHUNKS, CHUNK)
    dst2 = dst.reshape(NCHUNKS, CHUNK)
    w2 = w.reshape(NCHUNKS, CHUNK)

    @functools.partial(
        pl.kernel,
        out_type=jax.ShapeDtypeStruct((NC, N, f), jnp.float32),
        mesh=mesh,
        scratch_types=[
            pltpu.VMEM((CPT_LAST, CHUNK), jnp.int32),
            pltpu.VMEM((CPT_LAST, CHUNK), jnp.int32),
            pltpu.VMEM((CPT_LAST, CHUNK), jnp.float32),
            pltpu.VMEM((2, CHUNK, f), jnp.float32),
            pltpu.VMEM_SHARED((N, f), jnp.float32),
            pltpu.SemaphoreType.DMA((2,)),
        ],
        compiler_params=pltpu.CompilerParams(use_tc_tiling_on_sc=False),
    )
    def k(y_hbm, src_hbm, dst_hbm, w_hbm, z_hbm, out_hbm,
          sbuf, dbuf, wbuf, rows, acc, sems):
        cid = lax.axis_index("c")
        sid = lax.axis_index("s")
        wid = sid * NC + cid
        nct = jnp.where(wid == NW - 1, CPT_LAST, CPT)
        cstart = wid * CPT

        pltpu.sync_copy(
            z_hbm.at[pl.ds(sid * RPT, RPT)],
            acc.at[pl.ds(sid * RPT, RPT)],
        )

        @pl.when(sid == NS - 1)
        def _():
            pltpu.sync_copy(
                z_hbm.at[pl.ds(TAIL_BASE, TAIL)],
                acc.at[pl.ds(TAIL_BASE, TAIL)],
            )

        pltpu.sync_copy(src_hbm.at[pl.ds(cstart, CPT_LAST)], sbuf)
        pltpu.sync_copy(dst_hbm.at[pl.ds(cstart, CPT_LAST)], dbuf)
        pltpu.sync_copy(w_hbm.at[pl.ds(cstart, CPT_LAST)], wbuf)
        plsc.subcore_barrier()

        pltpu.async_copy(y_hbm.at[sbuf.at[0]], rows.at[0], sems.at[0])

        def chunk_body(c, carry):
            p = lax.rem(c, 2)

            @pl.when(c + 1 < nct)
            def _():
                pn = lax.rem(c + 1, 2)
                pltpu.async_copy(y_hbm.at[sbuf.at[c + 1]], rows.at[pn],
                                 sems.at[pn])

            pltpu.make_async_copy(y_hbm.at[sbuf.at[c]], rows.at[p],
                                  sems.at[p]).wait()

            for g in range(CHUNK // LANES):
                wv = wbuf[c, pl.ds(g * LANES, LANES)]
                for l in range(LANES):
                    wl = wv[l]
                    j = g * LANES + l
                    for fb in range(f // LANES):
                        v = rows[p, j, pl.ds(fb * LANES, LANES)]
                        rows[p, j, pl.ds(fb * LANES, LANES)] = v * wl

            pltpu.sync_copy(rows.at[p], acc.at[dbuf.at[c]], add=True)
            return carry

        lax.fori_loop(0, nct, chunk_body, 0)
        plsc.subcore_barrier()

        pltpu.sync_copy(
            acc.at[pl.ds(sid * RPT, RPT)],
            out_hbm.at[cid, pl.ds(sid * RPT, RPT)],
        )

        @pl.when(sid == NS - 1)
        def _():
            pltpu.sync_copy(
                acc.at[pl.ds(TAIL_BASE, TAIL)],
                out_hbm.at[cid, pl.ds(TAIL_BASE, TAIL)],
            )

    return k(y, src2, dst2, w2, zeros)


def _mm1(x, w1):
    bm = 1000

    def body(x_ref, w_ref, o_ref):
        o_ref[...] = jnp.dot(x_ref[...], w_ref[...],
                             preferred_element_type=jnp.float32)

    return pl.pallas_call(
        body,
        grid=(N // bm,),
        in_specs=[
            pl.BlockSpec((bm, D_IN), lambda i: (i, 0)),
            pl.BlockSpec((D_IN, H1), lambda i: (0, 0)),
        ],
        out_specs=pl.BlockSpec((bm, H1), lambda i: (i, 0)),
        out_shape=jax.ShapeDtypeStruct((N, H1), jnp.float32),
    )(x, w1)


def _relu_mm(p, w):
    fin = p.shape[2]
    fout = w.shape[1]
    bm = 2000

    def body(p_ref, w_ref, o_ref):
        h = jax.nn.relu(p_ref[0] + p_ref[1])
        o_ref[...] = jnp.dot(h, w_ref[...], preferred_element_type=jnp.float32)

    return pl.pallas_call(
        body,
        grid=(N // bm,),
        in_specs=[
            pl.BlockSpec((2, bm, fin), lambda i: (0, i, 0)),
            pl.BlockSpec((fin, fout), lambda i: (0, 0)),
        ],
        out_specs=pl.BlockSpec((bm, fout), lambda i: (i, 0)),
        out_shape=jax.ShapeDtypeStruct((N, fout), jnp.float32),
    )(p, w)


def _zsum(p):
    f = p.shape[2]
    bm = 2000

    def body(p_ref, o_ref):
        o_ref[...] = p_ref[0] + p_ref[1]

    return pl.pallas_call(
        body,
        grid=(N // bm,),
        in_specs=[pl.BlockSpec((2, bm, f), lambda i: (0, i, 0))],
        out_specs=pl.BlockSpec((bm, f), lambda i: (i, 0)),
        out_shape=jax.ShapeDtypeStruct((N, f), jnp.float32),
    )(p)


def _decode(z, zt):
    bm = 400

    def body(zr_ref, zc_ref, o_ref):
        logits = jnp.dot(zr_ref[...], zc_ref[...],
                         preferred_element_type=jnp.float32)
        o_ref[...] = jax.nn.sigmoid(logits)

    return pl.pallas_call(
        body,
        grid=(N // bm,),
        in_specs=[
            pl.BlockSpec((bm, H2), lambda i: (i, 0)),
            pl.BlockSpec((H2, N), lambda i: (0, 0)),
        ],
        out_specs=pl.BlockSpec((bm, N), lambda i: (i, 0)),
        out_shape=jax.ShapeDtypeStruct((N, N), jnp.float32),
    )(z, zt)


def kernel(X, edge_index, edge_weight, W1, W2, W3):
    src = edge_index[1]
    dst = edge_index[0]
    y1 = _mm1(X, W1)
    p1 = _spmm_sc(y1, src, dst, edge_weight)
    y2 = _relu_mm(p1, W2)
    p2 = _spmm_sc(y2, src, dst, edge_weight)
    y3 = _relu_mm(p2, W3)
    p3 = _spmm_sc(y3, src, dst, edge_weight)
    z = _zsum(p3)
    a = _decode(z, z.T)
    return (a, z)


# speedup vs baseline: 7.3403x; 1.0396x over previous
import functools

import jax
import jax.numpy as jnp
from jax import lax
from jax.experimental import pallas as pl
from jax.experimental.pallas import tpu as pltpu
from jax.experimental.pallas import tpu_sc as plsc

N = 10000
E = 160000
D_IN = 1433
H1 = 32
H2 = 16

NC = 2
NS = 16
NW = NC * NS
LANES = 16
CHUNK = 128
NCHUNKS = E // CHUNK
CPT = NCHUNKS // NW
CPT_LAST = NCHUNKS - (NW - 1) * CPT
RPT = 624
TAIL_BASE = NS * RPT
TAIL = N - TAIL_BASE


def _spmm_sc(y, ei3, w2):
    f = y.shape[1]
    mesh = plsc.VectorSubcoreMesh(
        core_axis_name="c", subcore_axis_name="s", num_cores=NC, num_subcores=NS
    )
    zeros = jnp.zeros((N, f), jnp.float32)

    @functools.partial(
        pl.kernel,
        out_type=jax.ShapeDtypeStruct((NC, N, f), jnp.float32),
        mesh=mesh,
        scratch_types=[
            pltpu.VMEM((CPT_LAST, CHUNK), jnp.int32),
            pltpu.VMEM((CPT_LAST, CHUNK), jnp.int32),
            pltpu.VMEM((CPT_LAST, CHUNK), jnp.float32),
            pltpu.VMEM((3, CHUNK, f), jnp.float32),
            pltpu.VMEM_SHARED((N, f), jnp.float32),
            pltpu.SemaphoreType.DMA((3,)),
            pltpu.SemaphoreType.DMA((3,)),
        ],
        compiler_params=pltpu.CompilerParams(use_tc_tiling_on_sc=False),
    )
    def k(y_hbm, ei_hbm, w_hbm, z_hbm, out_hbm,
          sbuf, dbuf, wbuf, rows, acc, gsems, ssems):
        cid = lax.axis_index("c")
        sid = lax.axis_index("s")
        wid = sid * NC + cid
        nct = jnp.where(wid == NW - 1, CPT_LAST, CPT)
        cstart = wid * CPT

        pltpu.sync_copy(
            z_hbm.at[pl.ds(sid * RPT, RPT)],
            acc.at[pl.ds(sid * RPT, RPT)],
        )

        @pl.when(sid == NS - 1)
        def _():
            pltpu.sync_copy(
                z_hbm.at[pl.ds(TAIL_BASE, TAIL)],
                acc.at[pl.ds(TAIL_BASE, TAIL)],
            )

        pltpu.sync_copy(ei_hbm.at[1, pl.ds(cstart, CPT_LAST)], sbuf)
        pltpu.sync_copy(ei_hbm.at[0, pl.ds(cstart, CPT_LAST)], dbuf)
        pltpu.sync_copy(w_hbm.at[pl.ds(cstart, CPT_LAST)], wbuf)
        plsc.subcore_barrier()

        pltpu.async_copy(y_hbm.at[sbuf.at[0]], rows.at[0], gsems.at[0])

        def chunk_body(c, carry):
            b = lax.rem(c, 3)

            @pl.when(c + 1 < nct)
            def _():
                bn = lax.rem(c + 1, 3)

                @pl.when(c >= 2)
                def _():
                    pltpu.make_async_copy(
                        rows.at[bn], acc.at[dbuf.at[c - 2]], ssems.at[bn]
                    ).wait()

                pltpu.async_copy(y_hbm.at[sbuf.at[c + 1]], rows.at[bn],
                                 gsems.at[bn])

            pltpu.make_async_copy(y_hbm.at[sbuf.at[c]], rows.at[b],
                                  gsems.at[b]).wait()

            for g in range(CHUNK // LANES):
                wv = wbuf[c, pl.ds(g * LANES, LANES)]
                for l in range(LANES):
                    wl = wv[l]
                    j = g * LANES + l
                    for fb in range(f // LANES):
                        v = rows[b, j, pl.ds(fb * LANES, LANES)]
                        rows[b, j, pl.ds(fb * LANES, LANES)] = v * wl

            pltpu.async_copy(rows.at[b], acc.at[dbuf.at[c]], ssems.at[b],
                             add=True)
            return carry

        lax.fori_loop(0, nct, chunk_body, 0)

        def drain_body(t, carry):
            cc = nct - 3 + t

            @pl.when(cc >= 0)
            def _():
                bb = lax.rem(cc, 3)
                pltpu.make_async_copy(
                    rows.at[bb], acc.at[dbuf.at[cc]], ssems.at[bb]
                ).wait()

            return carry

        lax.fori_loop(0, 3, drain_body, 0)
        plsc.subcore_barrier()

        pltpu.sync_copy(
            acc.at[pl.ds(sid * RPT, RPT)],
            out_hbm.at[cid, pl.ds(sid * RPT, RPT)],
        )

        @pl.when(sid == NS - 1)
        def _():
            pltpu.sync_copy(
                acc.at[pl.ds(TAIL_BASE, TAIL)],
                out_hbm.at[cid, pl.ds(TAIL_BASE, TAIL)],
            )

    return k(y, ei3, w2, zeros)


def _mm1(x, w1):
    bm = 1000

    def body(x_ref, w_ref, o_ref):
        o_ref[...] = jnp.dot(x_ref[...], w_ref[...],
                             preferred_element_type=jnp.float32)

    return pl.pallas_call(
        body,
        grid=(N // bm,),
        in_specs=[
            pl.BlockSpec((bm, D_IN), lambda i: (i, 0)),
            pl.BlockSpec((D_IN, H1), lambda i: (0, 0)),
        ],
        out_specs=pl.BlockSpec((bm, H1), lambda i: (i, 0)),
        out_shape=jax.ShapeDtypeStruct((N, H1), jnp.float32),
    )(x, w1)


def _relu_mm(p, w):
    fin = p.shape[2]
    fout = w.shape[1]
    bm = 2000

    def body(p_ref, w_ref, o_ref):
        h = jax.nn.relu(p_ref[0] + p_ref[1])
        o_ref[...] = jnp.dot(h, w_ref[...], preferred_element_type=jnp.float32)

    return pl.pallas_call(
        body,
        grid=(N // bm,),
        in_specs=[
            pl.BlockSpec((2, bm, fin), lambda i: (0, i, 0)),
            pl.BlockSpec((fin, fout), lambda i: (0, 0)),
        ],
        out_specs=pl.BlockSpec((bm, fout), lambda i: (i, 0)),
        out_shape=jax.ShapeDtypeStruct((N, fout), jnp.float32),
    )(p, w)


def _zsum(p):
    f = p.shape[2]

    def body(p_ref, o_ref, ot_ref):
        h = p_ref[0] + p_ref[1]
        o_ref[...] = h
        ot_ref[...] = h.T

    return pl.pallas_call(
        body,
        out_shape=[
            jax.ShapeDtypeStruct((N, f), jnp.float32),
            jax.ShapeDtypeStruct((f, N), jnp.float32),
        ],
    )(p)


def _decode(z, zt):
    bm = 400

    def body(zr_ref, zc_ref, o_ref):
        logits = jnp.dot(zr_ref[...], zc_ref[...],
                         preferred_element_type=jnp.float32)
        o_ref[...] = jax.nn.sigmoid(logits)

    return pl.pallas_call(
        body,
        grid=(N // bm,),
        in_specs=[
            pl.BlockSpec((bm, H2), lambda i: (i, 0)),
            pl.BlockSpec((H2, N), lambda i: (0, 0)),
        ],
        out_specs=pl.BlockSpec((bm, N), lambda i: (i, 0)),
        out_shape=jax.ShapeDtypeStruct((N, N), jnp.float32),
    )(z, zt)


def kernel(X, edge_index, edge_weight, W1, W2, W3):
    ei3 = edge_index.reshape(2, NCHUNKS, CHUNK)
    w2 = edge_weight.reshape(NCHUNKS, CHUNK)
    y1 = _mm1(X, W1)
    p1 = _spmm_sc(y1, ei3, w2)
    y2 = _relu_mm(p1, W2)
    p2 = _spmm_sc(y2, ei3, w2)
    y3 = _relu_mm(p2, W3)
    p3 = _spmm_sc(y3, ei3, w2)
    z, zt = _zsum(p3)
    a = _decode(z, zt)
    return (a, z)


# speedup vs baseline: 8.4308x; 1.1486x over previous
import functools

import jax
import jax.numpy as jnp
from jax import lax
from jax.experimental import pallas as pl
from jax.experimental.pallas import tpu as pltpu
from jax.experimental.pallas import tpu_sc as plsc

N = 10000
E = 160000
D_IN = 1433
H1 = 32
H2 = 16

NC = 2
NS = 16
NW = NC * NS
LANES = 16
CHUNK = 128
NCHUNKS = E // CHUNK
CPT = NCHUNKS // NW
CPT_LAST = NCHUNKS - (NW - 1) * CPT
RPT = 624
TAIL_BASE = NS * RPT
TAIL = N - TAIL_BASE


def _spmm_sc(y, ei3, w2):
    f = y.shape[1]
    mesh = plsc.VectorSubcoreMesh(
        core_axis_name="c", subcore_axis_name="s", num_cores=NC, num_subcores=NS
    )
    zeros = jnp.zeros((N, f), jnp.float32)

    @functools.partial(
        pl.kernel,
        out_type=jax.ShapeDtypeStruct((NC, N, f), jnp.float32),
        mesh=mesh,
        scratch_types=[
            pltpu.VMEM((CPT_LAST, CHUNK), jnp.int32),
            pltpu.VMEM((CPT_LAST, CHUNK), jnp.int32),
            pltpu.VMEM((CPT_LAST, CHUNK), jnp.float32),
            pltpu.VMEM((3, CHUNK, f), jnp.float32),
            pltpu.VMEM_SHARED((N, f), jnp.float32),
            pltpu.SemaphoreType.DMA((3,)),
            pltpu.SemaphoreType.DMA((3,)),
        ],
        compiler_params=pltpu.CompilerParams(use_tc_tiling_on_sc=False),
    )
    def k(y_hbm, ei_hbm, w_hbm, z_hbm, out_hbm,
          sbuf, dbuf, wbuf, rows, acc, gsems, ssems):
        cid = lax.axis_index("c")
        sid = lax.axis_index("s")
        wid = sid * NC + cid
        nct = jnp.where(wid == NW - 1, CPT_LAST, CPT)
        cstart = wid * CPT

        pltpu.sync_copy(
            z_hbm.at[pl.ds(sid * RPT, RPT)],
            acc.at[pl.ds(sid * RPT, RPT)],
        )

        @pl.when(sid == NS - 1)
        def _():
            pltpu.sync_copy(
                z_hbm.at[pl.ds(TAIL_BASE, TAIL)],
                acc.at[pl.ds(TAIL_BASE, TAIL)],
            )

        pltpu.sync_copy(ei_hbm.at[1, pl.ds(cstart, CPT_LAST)], sbuf)
        pltpu.sync_copy(ei_hbm.at[0, pl.ds(cstart, CPT_LAST)], dbuf)
        pltpu.sync_copy(w_hbm.at[pl.ds(cstart, CPT_LAST)], wbuf)
        plsc.subcore_barrier()

        pltpu.async_copy(y_hbm.at[sbuf.at[0]], rows.at[0], gsems.at[0])

        def chunk_body(c, carry):
            b = lax.rem(c, 3)

            @pl.when(c + 1 < nct)
            def _():
                bn = lax.rem(c + 1, 3)

                @pl.when(c >= 2)
                def _():
                    pltpu.make_async_copy(
                        rows.at[bn], acc.at[dbuf.at[c - 2]], ssems.at[bn]
                    ).wait()

                pltpu.async_copy(y_hbm.at[sbuf.at[c + 1]], rows.at[bn],
                                 gsems.at[bn])

            pltpu.make_async_copy(y_hbm.at[sbuf.at[c]], rows.at[b],
                                  gsems.at[b]).wait()

            for g in range(CHUNK // LANES):
                wv = wbuf[c, pl.ds(g * LANES, LANES)]
                for l in range(LANES):
                    wl = wv[l]
                    j = g * LANES + l
                    for fb in range(f // LANES):
                        v = rows[b, j, pl.ds(fb * LANES, LANES)]
                        rows[b, j, pl.ds(fb * LANES, LANES)] = v * wl

            pltpu.async_copy(rows.at[b], acc.at[dbuf.at[c]], ssems.at[b],
                             add=True)
            return carry

        lax.fori_loop(0, nct, chunk_body, 0)

        def drain_body(t, carry):
            cc = nct - 3 + t

            @pl.when(cc >= 0)
            def _():
                bb = lax.rem(cc, 3)
                pltpu.make_async_copy(
                    rows.at[bb], acc.at[dbuf.at[cc]], ssems.at[bb]
                ).wait()

            return carry

        lax.fori_loop(0, 3, drain_body, 0)
        plsc.subcore_barrier()

        pltpu.sync_copy(
            acc.at[pl.ds(sid * RPT, RPT)],
            out_hbm.at[cid, pl.ds(sid * RPT, RPT)],
        )

        @pl.when(sid == NS - 1)
        def _():
            pltpu.sync_copy(
                acc.at[pl.ds(TAIL_BASE, TAIL)],
                out_hbm.at[cid, pl.ds(TAIL_BASE, TAIL)],
            )

    return k(y, ei3, w2, zeros)


def _mm1(x, w1):
    xt = x.T
    bm = 1024

    def body(xt_ref, w_ref, o_ref):
        o_ref[...] = jax.lax.dot_general(
            xt_ref[...], w_ref[...], (((0,), (0,)), ((), ())),
            preferred_element_type=jnp.float32)

    return pl.pallas_call(
        body,
        grid=((N + bm - 1) // bm,),
        in_specs=[
            pl.BlockSpec((D_IN, bm), lambda i: (0, i)),
            pl.BlockSpec((D_IN, H1), lambda i: (0, 0)),
        ],
        out_specs=pl.BlockSpec((bm, H1), lambda i: (i, 0)),
        out_shape=jax.ShapeDtypeStruct((N, H1), jnp.float32),
    )(xt, w1)


def _relu_mm(p, w):
    fin = p.shape[2]
    fout = w.shape[1]
    bm = 2000

    def body(p_ref, w_ref, o_ref):
        h = jax.nn.relu(p_ref[0] + p_ref[1])
        o_ref[...] = jnp.dot(h, w_ref[...], preferred_element_type=jnp.float32)

    return pl.pallas_call(
        body,
        grid=(N // bm,),
        in_specs=[
            pl.BlockSpec((2, bm, fin), lambda i: (0, i, 0)),
            pl.BlockSpec((fin, fout), lambda i: (0, 0)),
        ],
        out_specs=pl.BlockSpec((bm, fout), lambda i: (i, 0)),
        out_shape=jax.ShapeDtypeStruct((N, fout), jnp.float32),
    )(p, w)


def _zsum(p):
    f = p.shape[2]

    def body(p_ref, o_ref, ot_ref):
        h = p_ref[0] + p_ref[1]
        o_ref[...] = h
        ot_ref[...] = h.T

    return pl.pallas_call(
        body,
        out_shape=[
            jax.ShapeDtypeStruct((N, f), jnp.float32),
            jax.ShapeDtypeStruct((f, N), jnp.float32),
        ],
    )(p)


def _decode(z, zt):
    bm = 400

    def body(zr_ref, zc_ref, o_ref):
        logits = jnp.dot(zr_ref[...], zc_ref[...],
                         preferred_element_type=jnp.float32)
        o_ref[...] = jax.nn.sigmoid(logits)

    return pl.pallas_call(
        body,
        grid=(N // bm,),
        in_specs=[
            pl.BlockSpec((bm, H2), lambda i: (i, 0)),
            pl.BlockSpec((H2, N), lambda i: (0, 0)),
        ],
        out_specs=pl.BlockSpec((bm, N), lambda i: (i, 0)),
        out_shape=jax.ShapeDtypeStruct((N, N), jnp.float32),
    )(z, zt)


def kernel(X, edge_index, edge_weight, W1, W2, W3):
    ei3 = edge_index.reshape(2, NCHUNKS, CHUNK)
    w2 = edge_weight.reshape(NCHUNKS, CHUNK)
    y1 = _mm1(X, W1)
    p1 = _spmm_sc(y1, ei3, w2)
    y2 = _relu_mm(p1, W2)
    p2 = _spmm_sc(y2, ei3, w2)
    y3 = _relu_mm(p2, W3)
    p3 = _spmm_sc(y3, ei3, w2)
    z, zt = _zsum(p3)
    a = _decode(z, zt)
    return (a, z)


# speedup vs baseline: 9.3802x; 1.1126x over previous
import functools

import jax
import jax.numpy as jnp
from jax import lax
from jax.experimental import pallas as pl
from jax.experimental.pallas import tpu as pltpu
from jax.experimental.pallas import tpu_sc as plsc

N = 10000
E = 160000
D_IN = 1433
H1 = 32
H2 = 16

NC = 2
NS = 16
NW = NC * NS
LANES = 16
CHUNK = 128
NCHUNKS = E // CHUNK
CPT = NCHUNKS // NW
CPT_LAST = NCHUNKS - (NW - 1) * CPT
RPT = 624
TAIL_BASE = NS * RPT
TAIL = N - TAIL_BASE


def _spmm_sc(y, ei3, w2):
    f = y.shape[1]
    mesh = plsc.VectorSubcoreMesh(
        core_axis_name="c", subcore_axis_name="s", num_cores=NC, num_subcores=NS
    )
    zeros = jnp.zeros((N, f), jnp.float32)

    @functools.partial(
        pl.kernel,
        out_type=jax.ShapeDtypeStruct((NC, N, f), jnp.float32),
        mesh=mesh,
        scratch_types=[
            pltpu.VMEM((CPT_LAST, CHUNK), jnp.int32),
            pltpu.VMEM((CPT_LAST, CHUNK), jnp.int32),
            pltpu.VMEM((CPT_LAST, CHUNK), jnp.float32),
            pltpu.VMEM((5, CHUNK, f), jnp.float32),
            pltpu.VMEM_SHARED((N, f), jnp.float32),
            pltpu.SemaphoreType.DMA((5,)),
            pltpu.SemaphoreType.DMA((5,)),
        ],
        compiler_params=pltpu.CompilerParams(use_tc_tiling_on_sc=False),
    )
    def k(y_hbm, ei_hbm, w_hbm, z_hbm, out_hbm,
          sbuf, dbuf, wbuf, rows, acc, gsems, ssems):
        cid = lax.axis_index("c")
        sid = lax.axis_index("s")
        wid = sid * NC + cid
        nct = jnp.where(wid == NW - 1, CPT_LAST, CPT)
        cstart = wid * CPT

        pltpu.sync_copy(
            z_hbm.at[pl.ds(sid * RPT, RPT)],
            acc.at[pl.ds(sid * RPT, RPT)],
        )

        @pl.when(sid == NS - 1)
        def _():
            pltpu.sync_copy(
                z_hbm.at[pl.ds(TAIL_BASE, TAIL)],
                acc.at[pl.ds(TAIL_BASE, TAIL)],
            )

        pltpu.sync_copy(ei_hbm.at[1, pl.ds(cstart, CPT_LAST)], sbuf)
        pltpu.sync_copy(ei_hbm.at[0, pl.ds(cstart, CPT_LAST)], dbuf)
        pltpu.sync_copy(w_hbm.at[pl.ds(cstart, CPT_LAST)], wbuf)
        plsc.subcore_barrier()

        for c0 in range(3):
            @pl.when(c0 < nct)
            def _():
                pltpu.async_copy(y_hbm.at[sbuf.at[c0]], rows.at[c0],
                                 gsems.at[c0])

        def chunk_body(c, carry):
            b = lax.rem(c, 5)

            @pl.when(c + 3 < nct)
            def _():
                bn = lax.rem(c + 3, 5)

                @pl.when(c >= 2)
                def _():
                    pltpu.make_async_copy(
                        rows.at[bn], acc.at[dbuf.at[c - 2]], ssems.at[bn]
                    ).wait()

                pltpu.async_copy(y_hbm.at[sbuf.at[c + 3]], rows.at[bn],
                                 gsems.at[bn])

            pltpu.make_async_copy(y_hbm.at[sbuf.at[c]], rows.at[b],
                                  gsems.at[b]).wait()

            for g in range(CHUNK // LANES):
                wv = wbuf[c, pl.ds(g * LANES, LANES)]
                for l in range(LANES):
                    wl = wv[l]
                    j = g * LANES + l
                    for fb in range(f // LANES):
                        v = rows[b, j, pl.ds(fb * LANES, LANES)]
                        rows[b, j, pl.ds(fb * LANES, LANES)] = v * wl

            pltpu.async_copy(rows.at[b], acc.at[dbuf.at[c]], ssems.at[b],
                             add=True)
            return carry

        lax.fori_loop(0, nct, chunk_body, 0)

        def drain_body(t, carry):
            cc = nct - 5 + t

            @pl.when(cc >= 0)
            def _():
                bb = lax.rem(cc, 5)
                pltpu.make_async_copy(
                    rows.at[bb], acc.at[dbuf.at[cc]], ssems.at[bb]
                ).wait()

            return carry

        lax.fori_loop(0, 5, drain_body, 0)
        plsc.subcore_barrier()

        pltpu.sync_copy(
            acc.at[pl.ds(sid * RPT, RPT)],
            out_hbm.at[cid, pl.ds(sid * RPT, RPT)],
        )

        @pl.when(sid == NS - 1)
        def _():
            pltpu.sync_copy(
                acc.at[pl.ds(TAIL_BASE, TAIL)],
                out_hbm.at[cid, pl.ds(TAIL_BASE, TAIL)],
            )

    return k(y, ei3, w2, zeros)


def _mm1(x, w1):
    xt = x.T
    bm = 2048

    def body(xt_ref, w_ref, o_ref):
        o_ref[...] = jax.lax.dot_general(
            xt_ref[...], w_ref[...], (((0,), (0,)), ((), ())),
            preferred_element_type=jnp.float32)

    return pl.pallas_call(
        body,
        grid=((N + bm - 1) // bm,),
        in_specs=[
            pl.BlockSpec((D_IN, bm), lambda i: (0, i)),
            pl.BlockSpec((D_IN, H1), lambda i: (0, 0)),
        ],
        out_specs=pl.BlockSpec((bm, H1), lambda i: (i, 0)),
        out_shape=jax.ShapeDtypeStruct((N, H1), jnp.float32),
    )(xt, w1)


def _relu_mm(p, w):
    fin = p.shape[2]
    fout = w.shape[1]
    bm = 2000

    def body(p_ref, w_ref, o_ref):
        h = jax.nn.relu(p_ref[0] + p_ref[1])
        o_ref[...] = jnp.dot(h, w_ref[...], preferred_element_type=jnp.float32)

    return pl.pallas_call(
        body,
        grid=(N // bm,),
        in_specs=[
            pl.BlockSpec((2, bm, fin), lambda i: (0, i, 0)),
            pl.BlockSpec((fin, fout), lambda i: (0, 0)),
        ],
        out_specs=pl.BlockSpec((bm, fout), lambda i: (i, 0)),
        out_shape=jax.ShapeDtypeStruct((N, fout), jnp.float32),
    )(p, w)


def _zsum(p):
    f = p.shape[2]

    def body(p_ref, o_ref, ot_ref):
        h = p_ref[0] + p_ref[1]
        o_ref[...] = h
        ot_ref[...] = h.T

    return pl.pallas_call(
        body,
        out_shape=[
            jax.ShapeDtypeStruct((N, f), jnp.float32),
            jax.ShapeDtypeStruct((f, N), jnp.float32),
        ],
    )(p)


def _decode(z, zt):
    bm = 400

    def body(zr_ref, zc_ref, o_ref):
        logits = jnp.dot(zr_ref[...], zc_ref[...],
                         preferred_element_type=jnp.float32)
        o_ref[...] = 0.5 * jnp.tanh(0.5 * logits) + 0.5

    return pl.pallas_call(
        body,
        grid=(N // bm,),
        in_specs=[
            pl.BlockSpec((bm, H2), lambda i: (i, 0)),
            pl.BlockSpec((H2, N), lambda i: (0, 0)),
        ],
        out_specs=pl.BlockSpec((bm, N), lambda i: (i, 0)),
        out_shape=jax.ShapeDtypeStruct((N, N), jnp.float32),
    )(z, zt)


def kernel(X, edge_index, edge_weight, W1, W2, W3):
    ei3 = edge_index.reshape(2, NCHUNKS, CHUNK)
    w2 = edge_weight.reshape(NCHUNKS, CHUNK)
    y1 = _mm1(X, W1)
    p1 = _spmm_sc(y1, ei3, w2)
    y2 = _relu_mm(p1, W2)
    p2 = _spmm_sc(y2, ei3, w2)
    y3 = _relu_mm(p2, W3)
    p3 = _spmm_sc(y3, ei3, w2)
    z, zt = _zsum(p3)
    a = _decode(z, zt)
    return (a, z)


# speedup vs baseline: 10.1790x; 1.0852x over previous
import functools

import jax
import jax.numpy as jnp
from jax import lax
from jax.experimental import pallas as pl
from jax.experimental.pallas import tpu as pltpu
from jax.experimental.pallas import tpu_sc as plsc

N = 10000
E = 160000
D_IN = 1433
H1 = 32
H2 = 16

NC = 2
NS = 16
NW = NC * NS
LANES = 16
CHUNK = 128
NCHUNKS = E // CHUNK
CPT = NCHUNKS // NW
CPT_LAST = NCHUNKS - (NW - 1) * CPT
RPT = 624
TAIL_BASE = NS * RPT
TAIL = N - TAIL_BASE


def _spmm_sc(y, ei3, w2):
    f = y.shape[1]
    mesh = plsc.VectorSubcoreMesh(
        core_axis_name="c", subcore_axis_name="s", num_cores=NC, num_subcores=NS
    )
    zeros = jnp.zeros((N, f), jnp.float32)

    @functools.partial(
        pl.kernel,
        out_type=jax.ShapeDtypeStruct((NC, N, f), jnp.float32),
        mesh=mesh,
        scratch_types=[
            pltpu.VMEM((CPT_LAST, CHUNK), jnp.int32),
            pltpu.VMEM((CPT_LAST, CHUNK), jnp.int32),
            pltpu.VMEM((CPT_LAST, CHUNK), jnp.float32),
            pltpu.VMEM((5, CHUNK, f), jnp.float32),
            pltpu.VMEM_SHARED((N, f), jnp.float32),
            pltpu.SemaphoreType.DMA((5,)),
            pltpu.SemaphoreType.DMA((5,)),
        ],
        compiler_params=pltpu.CompilerParams(use_tc_tiling_on_sc=False),
    )
    def k(y_hbm, ei_hbm, w_hbm, z_hbm, out_hbm,
          sbuf, dbuf, wbuf, rows, acc, gsems, ssems):
        cid = lax.axis_index("c")
        sid = lax.axis_index("s")
        wid = sid * NC + cid
        nct = jnp.where(wid == NW - 1, CPT_LAST, CPT)
        cstart = wid * CPT

        pltpu.sync_copy(
            z_hbm.at[pl.ds(sid * RPT, RPT)],
            acc.at[pl.ds(sid * RPT, RPT)],
        )

        @pl.when(sid == NS - 1)
        def _():
            pltpu.sync_copy(
                z_hbm.at[pl.ds(TAIL_BASE, TAIL)],
                acc.at[pl.ds(TAIL_BASE, TAIL)],
            )

        pltpu.sync_copy(ei_hbm.at[1, pl.ds(cstart, CPT_LAST)], sbuf)
        pltpu.sync_copy(ei_hbm.at[0, pl.ds(cstart, CPT_LAST)], dbuf)
        pltpu.sync_copy(w_hbm.at[pl.ds(cstart, CPT_LAST)], wbuf)
        plsc.subcore_barrier()

        for c0 in range(3):
            @pl.when(c0 < nct)
            def _():
                pltpu.async_copy(y_hbm.at[sbuf.at[c0]], rows.at[c0],
                                 gsems.at[c0])

        def chunk_body(c, carry):
            b = lax.rem(c, 5)

            @pl.when(c + 3 < nct)
            def _():
                bn = lax.rem(c + 3, 5)

                @pl.when(c >= 2)
                def _():
                    pltpu.make_async_copy(
                        rows.at[bn], acc.at[dbuf.at[c - 2]], ssems.at[bn]
                    ).wait()

                pltpu.async_copy(y_hbm.at[sbuf.at[c + 3]], rows.at[bn],
                                 gsems.at[bn])

            pltpu.make_async_copy(y_hbm.at[sbuf.at[c]], rows.at[b],
                                  gsems.at[b]).wait()

            for g in range(CHUNK // LANES):
                wv = wbuf[c, pl.ds(g * LANES, LANES)]
                for l in range(LANES):
                    wl = wv[l]
                    j = g * LANES + l
                    for fb in range(f // LANES):
                        v = rows[b, j, pl.ds(fb * LANES, LANES)]
                        rows[b, j, pl.ds(fb * LANES, LANES)] = v * wl

            pltpu.async_copy(rows.at[b], acc.at[dbuf.at[c]], ssems.at[b],
                             add=True)
            return carry

        lax.fori_loop(0, nct, chunk_body, 0)

        def drain_body(t, carry):
            cc = nct - 5 + t

            @pl.when(cc >= 0)
            def _():
                bb = lax.rem(cc, 5)
                pltpu.make_async_copy(
                    rows.at[bb], acc.at[dbuf.at[cc]], ssems.at[bb]
                ).wait()

            return carry

        lax.fori_loop(0, 5, drain_body, 0)
        plsc.subcore_barrier()

        pltpu.sync_copy(
            acc.at[pl.ds(sid * RPT, RPT)],
            out_hbm.at[cid, pl.ds(sid * RPT, RPT)],
        )

        @pl.when(sid == NS - 1)
        def _():
            pltpu.sync_copy(
                acc.at[pl.ds(TAIL_BASE, TAIL)],
                out_hbm.at[cid, pl.ds(TAIL_BASE, TAIL)],
            )

    return k(y, ei3, w2, zeros)


def _mm1(x, w1):
    xt = x.T
    bm = 2048

    def body(xt_ref, w_ref, o_ref):
        o_ref[...] = jax.lax.dot_general(
            xt_ref[...], w_ref[...], (((0,), (0,)), ((), ())),
            preferred_element_type=jnp.float32)

    return pl.pallas_call(
        body,
        grid=((N + bm - 1) // bm,),
        in_specs=[
            pl.BlockSpec((D_IN, bm), lambda i: (0, i)),
            pl.BlockSpec((D_IN, H1), lambda i: (0, 0)),
        ],
        out_specs=pl.BlockSpec((bm, H1), lambda i: (i, 0)),
        out_shape=jax.ShapeDtypeStruct((N, H1), jnp.float32),
    )(xt, w1)


def _relu_mm(p, w):
    fin = p.shape[2]
    fout = w.shape[1]
    rows_in = N * fin // 128
    rows_out = N * fout // 128
    kin = (128 * rows_in) // rows_out
    nrep = kin // fin
    wbig = jnp.kron(jnp.eye(nrep, dtype=jnp.float32), w)
    p2 = p.reshape(2, rows_out, kin)
    bm = 256

    def body(p_ref, w_ref, o_ref):
        h = jax.nn.relu(p_ref[0] + p_ref[1])
        o_ref[...] = jnp.dot(h, w_ref[...], preferred_element_type=jnp.float32)

    grid = (rows_out + bm - 1) // bm
    out2 = pl.pallas_call(
        body,
        grid=(grid,),
        in_specs=[
            pl.BlockSpec((2, bm, kin), lambda i: (0, i, 0)),
            pl.BlockSpec((kin, 128), lambda i: (0, 0)),
        ],
        out_specs=pl.BlockSpec((bm, 128), lambda i: (i, 0)),
        out_shape=jax.ShapeDtypeStruct((rows_out, 128), jnp.float32),
    )(p2, wbig)
    return out2.reshape(N, fout)


def _zsum(p):
    f = p.shape[2]

    def body(p_ref, o_ref, ot_ref):
        h = p_ref[0] + p_ref[1]
        o_ref[...] = h
        ot_ref[...] = h.T

    return pl.pallas_call(
        body,
        out_shape=[
            jax.ShapeDtypeStruct((N, f), jnp.float32),
            jax.ShapeDtypeStruct((f, N), jnp.float32),
        ],
    )(p)


def _decode(z, zt):
    bm = 400

    def body(zr_ref, zc_ref, o_ref):
        logits = jnp.dot(zr_ref[...], zc_ref[...],
                         preferred_element_type=jnp.float32)
        o_ref[...] = 0.5 * jnp.tanh(0.5 * logits) + 0.5

    return pl.pallas_call(
        body,
        grid=(N // bm,),
        in_specs=[
            pl.BlockSpec((bm, H2), lambda i: (i, 0)),
            pl.BlockSpec((H2, N), lambda i: (0, 0)),
        ],
        out_specs=pl.BlockSpec((bm, N), lambda i: (i, 0)),
        out_shape=jax.ShapeDtypeStruct((N, N), jnp.float32),
    )(z, zt)


def kernel(X, edge_index, edge_weight, W1, W2, W3):
    ei3 = edge_index.reshape(2, NCHUNKS, CHUNK)
    w2 = edge_weight.reshape(NCHUNKS, CHUNK)
    y1 = _mm1(X, W1)
    p1 = _spmm_sc(y1, ei3, w2)
    y2 = _relu_mm(p1, W2)
    p2 = _spmm_sc(y2, ei3, w2)
    y3 = _relu_mm(p2, W3)
    p3 = _spmm_sc(y3, ei3, w2)
    z, zt = _zsum(p3)
    a = _decode(z, zt)
    return (a, z)


# speedup vs baseline: 10.2869x; 1.0106x over previous
import functools

import jax
import jax.numpy as jnp
from jax import lax
from jax.experimental import pallas as pl
from jax.experimental.pallas import tpu as pltpu
from jax.experimental.pallas import tpu_sc as plsc

N = 10000
E = 160000
D_IN = 1433
H1 = 32
H2 = 16

NC = 2
NS = 16
NW = NC * NS
LANES = 16
CHUNK = 128
NCHUNKS = E // CHUNK
CPT = NCHUNKS // NW
CPT_LAST = NCHUNKS - (NW - 1) * CPT
RPT = 624
TAIL_BASE = NS * RPT
TAIL = N - TAIL_BASE


def _spmm_sc(y, ei3, w2):
    f = y.shape[1]
    mesh = plsc.VectorSubcoreMesh(
        core_axis_name="c", subcore_axis_name="s", num_cores=NC, num_subcores=NS
    )

    @functools.partial(
        pl.kernel,
        out_type=jax.ShapeDtypeStruct((NC, N, f), jnp.float32),
        mesh=mesh,
        scratch_types=[
            pltpu.VMEM((CPT_LAST, CHUNK), jnp.int32),
            pltpu.VMEM((CPT_LAST, CHUNK), jnp.int32),
            pltpu.VMEM((CPT_LAST, CHUNK), jnp.float32),
            pltpu.VMEM((5, CHUNK, f), jnp.float32),
            pltpu.VMEM_SHARED((N, f), jnp.float32),
            pltpu.SemaphoreType.DMA((5,)),
            pltpu.SemaphoreType.DMA((5,)),
        ],
        compiler_params=pltpu.CompilerParams(use_tc_tiling_on_sc=False),
    )
    def k(y_hbm, ei_hbm, w_hbm, out_hbm,
          sbuf, dbuf, wbuf, rows, acc, gsems, ssems):
        cid = lax.axis_index("c")
        sid = lax.axis_index("s")
        wid = sid * NC + cid
        nct = jnp.where(wid == NW - 1, CPT_LAST, CPT)
        cstart = wid * CPT

        def zbody(j, carry):
            for fb in range(f // LANES):
                rows[0, j, pl.ds(fb * LANES, LANES)] = jnp.zeros(
                    (LANES,), jnp.float32)
            return carry

        lax.fori_loop(0, CHUNK, zbody, 0)
        for t in range(RPT // CHUNK):
            pltpu.sync_copy(rows.at[0],
                            acc.at[pl.ds(sid * RPT + t * CHUNK, CHUNK)])
        _REM = RPT % CHUNK
        pltpu.sync_copy(
            rows.at[0, pl.ds(0, _REM)],
            acc.at[pl.ds(sid * RPT + (RPT // CHUNK) * CHUNK, _REM)],
        )

        @pl.when(sid == NS - 1)
        def _():
            pltpu.sync_copy(
                rows.at[0, pl.ds(0, TAIL)],
                acc.at[pl.ds(TAIL_BASE, TAIL)],
            )

        pltpu.sync_copy(ei_hbm.at[1, pl.ds(cstart, CPT_LAST)], sbuf)
        pltpu.sync_copy(ei_hbm.at[0, pl.ds(cstart, CPT_LAST)], dbuf)
        pltpu.sync_copy(w_hbm.at[pl.ds(cstart, CPT_LAST)], wbuf)
        plsc.subcore_barrier()

        for c0 in range(3):
            @pl.when(c0 < nct)
            def _():
                pltpu.async_copy(y_hbm.at[sbuf.at[c0]], rows.at[c0],
                                 gsems.at[c0])

        def chunk_body(c, carry):
            b = lax.rem(c, 5)

            @pl.when(c + 3 < nct)
            def _():
                bn = lax.rem(c + 3, 5)

                @pl.when(c >= 2)
                def _():
                    pltpu.make_async_copy(
                        rows.at[bn], acc.at[dbuf.at[c - 2]], ssems.at[bn]
                    ).wait()

                pltpu.async_copy(y_hbm.at[sbuf.at[c + 3]], rows.at[bn],
                                 gsems.at[bn])

            pltpu.make_async_copy(y_hbm.at[sbuf.at[c]], rows.at[b],
                                  gsems.at[b]).wait()

            for g in range(CHUNK // LANES):
                wv = wbuf[c, pl.ds(g * LANES, LANES)]
                for l in range(LANES):
                    wl = wv[l]
                    j = g * LANES + l
                    for fb in range(f // LANES):
                        v = rows[b, j, pl.ds(fb * LANES, LANES)]
                        rows[b, j, pl.ds(fb * LANES, LANES)] = v * wl

            pltpu.async_copy(rows.at[b], acc.at[dbuf.at[c]], ssems.at[b],
                             add=True)
            return carry

        lax.fori_loop(0, nct, chunk_body, 0)

        def drain_body(t, carry):
            cc = nct - 5 + t

            @pl.when(cc >= 0)
            def _():
                bb = lax.rem(cc, 5)
                pltpu.make_async_copy(
                    rows.at[bb], acc.at[dbuf.at[cc]], ssems.at[bb]
                ).wait()

            return carry

        lax.fori_loop(0, 5, drain_body, 0)
        plsc.subcore_barrier()

        pltpu.sync_copy(
            acc.at[pl.ds(sid * RPT, RPT)],
            out_hbm.at[cid, pl.ds(sid * RPT, RPT)],
        )

        @pl.when(sid == NS - 1)
        def _():
            pltpu.sync_copy(
                acc.at[pl.ds(TAIL_BASE, TAIL)],
                out_hbm.at[cid, pl.ds(TAIL_BASE, TAIL)],
            )

    return k(y, ei3, w2)


def _mm1(x, w1):
    xt = x.T
    bm = 2048

    def body(xt_ref, w_ref, o_ref):
        o_ref[...] = jax.lax.dot_general(
            xt_ref[...], w_ref[...], (((0,), (0,)), ((), ())),
            preferred_element_type=jnp.float32)

    return pl.pallas_call(
        body,
        grid=((N + bm - 1) // bm,),
        in_specs=[
            pl.BlockSpec((D_IN, bm), lambda i: (0, i)),
            pl.BlockSpec((D_IN, H1), lambda i: (0, 0)),
        ],
        out_specs=pl.BlockSpec((bm, H1), lambda i: (i, 0)),
        out_shape=jax.ShapeDtypeStruct((N, H1), jnp.float32),
    )(xt, w1)


def _relu_mm(p, w):
    fin = p.shape[2]
    fout = w.shape[1]
    rows_in = N * fin // 128
    rows_out = N * fout // 128
    kin = (128 * rows_in) // rows_out
    nrep = kin // fin
    wbig = jnp.kron(jnp.eye(nrep, dtype=jnp.float32), w)
    p2 = p.reshape(2, rows_out, kin)
    bm = 256

    def body(p_ref, w_ref, o_ref):
        h = jax.nn.relu(p_ref[0] + p_ref[1])
        o_ref[...] = jnp.dot(h, w_ref[...], preferred_element_type=jnp.float32)

    grid = (rows_out + bm - 1) // bm
    out2 = pl.pallas_call(
        body,
        grid=(grid,),
        in_specs=[
            pl.BlockSpec((2, bm, kin), lambda i: (0, i, 0)),
            pl.BlockSpec((kin, 128), lambda i: (0, 0)),
        ],
        out_specs=pl.BlockSpec((bm, 128), lambda i: (i, 0)),
        out_shape=jax.ShapeDtypeStruct((rows_out, 128), jnp.float32),
    )(p2, wbig)
    return out2.reshape(N, fout)


def _zsum(p):
    f = p.shape[2]

    def body(p_ref, o_ref, ot_ref):
        h = p_ref[0] + p_ref[1]
        o_ref[...] = h
        ot_ref[...] = h.T

    return pl.pallas_call(
        body,
        out_shape=[
            jax.ShapeDtypeStruct((N, f), jnp.float32),
            jax.ShapeDtypeStruct((f, N), jnp.float32),
        ],
    )(p)


def _decode(z, zt):
    bm = 400

    def body(zr_ref, zc_ref, o_ref):
        logits = jnp.dot(zr_ref[...], zc_ref[...],
                         preferred_element_type=jnp.float32)
        o_ref[...] = 0.5 * jnp.tanh(0.5 * logits) + 0.5

    return pl.pallas_call(
        body,
        grid=(N // bm,),
        in_specs=[
            pl.BlockSpec((bm, H2), lambda i: (i, 0)),
            pl.BlockSpec((H2, N), lambda i: (0, 0)),
        ],
        out_specs=pl.BlockSpec((bm, N), lambda i: (i, 0)),
        out_shape=jax.ShapeDtypeStruct((N, N), jnp.float32),
    )(z, zt)


def kernel(X, edge_index, edge_weight, W1, W2, W3):
    ei3 = edge_index.reshape(2, NCHUNKS, CHUNK)
    w2 = edge_weight.reshape(NCHUNKS, CHUNK)
    y1 = _mm1(X, W1)
    p1 = _spmm_sc(y1, ei3, w2)
    y2 = _relu_mm(p1, W2)
    p2 = _spmm_sc(y2, ei3, w2)
    y3 = _relu_mm(p2, W3)
    p3 = _spmm_sc(y3, ei3, w2)
    z, zt = _zsum(p3)
    a = _decode(z, zt)
    return (a, z)
